# Initial kernel scaffold; baseline (speedup 1.0000x reference)
import jax
import jax.numpy as jnp
from jax.experimental import pallas as pl


def kernel(pred_class, pred_bbox, gt_class, gt_bbox):
    raise NotImplementedError("write your pallas kernel here")


# speedup vs baseline: 4.8558x; 4.8558x over previous
import functools

import jax
import jax.numpy as jnp
from jax import lax
from jax.experimental import pallas as pl
from jax.experimental.pallas import tpu as pltpu
from jax.experimental.pallas import tpu_sc as plsc

NUM_CLASSES = 91
C1 = NUM_CLASSES + 1
EOS_COEF = 0.1
W_CLASS, W_BBOX, W_GIOU = 1.0, 5.0, 2.0
EPS = 1e-7
B, Q, G = 64, 300, 50
QG = Q * G
LANES = 16
NCHUNK = (Q + LANES - 1) // LANES
QPAD = NCHUNK * LANES


def _dense_body(pc_ref, pb_ref, gbt_ref, gc_ref, gcall_ref,
                cost_ref, lossp_ref, extra_ref):
    pc = pc_ref[0]
    pb = pb_ref[0]
    gbt = gbt_ref[0]
    gc = gc_ref[0]
    gcall = gcall_ref[...]

    m = jnp.max(pc, axis=1, keepdims=True)
    e = jnp.exp(pc - m)
    lse = jnp.log(jnp.sum(e, axis=1, keepdims=True)) + m
    logp0 = pc[:, 0:1] - lse

    cls_iota = lax.broadcasted_iota(jnp.int32, (C1, G), 0)
    oh = (cls_iota == gc).astype(jnp.float32)
    pcg = jnp.dot(pc, oh, preferred_element_type=jnp.float32)
    logp_g = pcg - lse
    prob_g = jnp.exp(logp_g)

    px0, py0, px1, py1 = pb[:, 0:1], pb[:, 1:2], pb[:, 2:3], pb[:, 3:4]
    gx0, gy0, gx1, gy1 = gbt[0:1, :], gbt[1:2, :], gbt[2:3, :], gbt[3:4, :]
    l1 = (jnp.abs(px0 - gx0) + jnp.abs(py0 - gy0)
          + jnp.abs(px1 - gx1) + jnp.abs(py1 - gy1))
    ix1 = jnp.maximum(px0, gx0)
    iy1 = jnp.maximum(py0, gy0)
    ix2 = jnp.minimum(px1, gx1)
    iy2 = jnp.minimum(py1, gy1)
    inter = jnp.maximum(ix2 - ix1, 0.0) * jnp.maximum(iy2 - iy1, 0.0)
    a1 = (px1 - px0) * (py1 - py0)
    a2 = (gx1 - gx0) * (gy1 - gy0)
    union = a1 + a2 - inter
    iou = inter / (union + EPS)
    ex1 = jnp.minimum(px0, gx0)
    ey1 = jnp.minimum(py0, gy0)
    ex2 = jnp.maximum(px1, gx1)
    ey2 = jnp.maximum(py1, gy1)
    enc = (ex2 - ex1) * (ey2 - ey1)
    giou = iou - (enc - union) / (enc + EPS)

    cost = -W_CLASS * prob_g + W_BBOX * l1 - W_GIOU * giou

    nvalid = jnp.sum((gcall != 0).astype(jnp.float32))
    class_den = EOS_COEF * (B * Q) + (1.0 - EOS_COEF) * nvalid
    num_objs = jnp.maximum(nvalid, 1.0)

    wg = jnp.where(gc == 0, EOS_COEF, 1.0).astype(jnp.float32)
    validg = (gc != 0).astype(jnp.float32)
    lossp = (W_CLASS * (wg * (-logp_g) + EOS_COEF * logp0) / class_den
             + validg * (W_BBOX * l1 + W_GIOU * (1.0 - giou)) / num_objs)

    cost_ref[0] = cost
    lossp_ref[0] = lossp
    extra_ref[0] = jnp.full((1, 1), W_CLASS * EOS_COEF, jnp.float32) * (
        -jnp.sum(logp0)) / class_den


def _dense_call(pred_class, pred_bbox, gt_bboxT, gt_class_r, gt_class):
    return pl.pallas_call(
        _dense_body,
        grid=(B,),
        in_specs=[
            pl.BlockSpec((1, Q, C1), lambda b: (b, 0, 0)),
            pl.BlockSpec((1, Q, 4), lambda b: (b, 0, 0)),
            pl.BlockSpec((1, 4, G), lambda b: (b, 0, 0)),
            pl.BlockSpec((1, 1, G), lambda b: (b, 0, 0)),
            pl.BlockSpec((B, G), lambda b: (0, 0)),
        ],
        out_specs=[
            pl.BlockSpec((1, Q, G), lambda b: (b, 0, 0)),
            pl.BlockSpec((1, Q, G), lambda b: (b, 0, 0)),
            pl.BlockSpec((1, 1, 1), lambda b: (b, 0, 0)),
        ],
        out_shape=[
            jax.ShapeDtypeStruct((B, Q, G), jnp.float32),
            jax.ShapeDtypeStruct((B, Q, G), jnp.float32),
            jax.ShapeDtypeStruct((B, 1, 1), jnp.float32),
        ],
    )(pred_class, pred_bbox, gt_bboxT, gt_class_r, gt_class)


@functools.lru_cache(maxsize=1)
def _get_match_kernel():
    mesh = plsc.VectorSubcoreMesh(core_axis_name="c", subcore_axis_name="s",
                                  num_cores=2, num_subcores=16)
    return functools.partial(
        pl.kernel,
        out_type=jax.ShapeDtypeStruct((B, LANES), jnp.float32),
        mesh=mesh,
        scratch_types=[
            pltpu.VMEM((QG,), jnp.float32),
            pltpu.VMEM((QG,), jnp.float32),
            pltpu.VMEM((QPAD,), jnp.float32),
            pltpu.VMEM((LANES,), jnp.float32),
        ],
        compiler_params=pltpu.CompilerParams(needs_layout_passes=False),
    )(_match_body)


def _match_body(cost_hbm, lossp_hbm, out_hbm, cost_v, lossp_v, pen_v, acc_v):
    wid = lax.axis_index("s") * 2 + lax.axis_index("c")
    lane = lax.iota(jnp.int32, LANES)
    lane0 = lane == 0
    idx_base = lane * G
    zeros = jnp.zeros((LANES,), jnp.float32)
    ones = jnp.ones((LANES,), jnp.float32)

    for i in range(2):
        b = wid * 2 + i
        pltpu.sync_copy(cost_hbm.at[b], cost_v)
        pltpu.sync_copy(lossp_hbm.at[b], lossp_v)
        for k in range(NCHUNK):
            pen_v[pl.ds(k * LANES, LANES)] = zeros

        def gbody(g, acc):
            mvec = jnp.full((LANES,), jnp.inf, jnp.float32)
            kidx = jnp.zeros((LANES,), jnp.int32)
            for k in range(NCHUNK):
                idx = idx_base + (g + k * LANES * G)
                idx = jnp.minimum(idx, QG - 1)
                vals = plsc.load_gather(cost_v, [idx])
                pen = pen_v[pl.ds(k * LANES, LANES)]
                aug = jnp.where(pen > 0.5, jnp.inf, vals)
                if k == NCHUNK - 1:
                    aug = jnp.where(lane < Q - (NCHUNK - 1) * LANES,
                                    aug, jnp.inf)
                better = aug < mvec
                mvec = jnp.where(better, aug, mvec)
                kidx = jnp.where(better, k, kidx)
            mval = jnp.min(mvec)
            cand = jnp.where(mvec == mval, kidx * LANES + lane,
                             jnp.int32(1 << 30))
            q = jnp.min(cand)
            plsc.store_scatter(pen_v, [jnp.full((LANES,), q, jnp.int32)],
                               ones, mask=lane0)
            li = jnp.full((LANES,), q * G + g, jnp.int32)
            lv = plsc.load_gather(lossp_v, [li])
            return acc + jnp.where(lane0, lv, 0.0)

        acc = lax.fori_loop(0, G, gbody, jnp.zeros((LANES,), jnp.float32))
        acc_v[...] = acc
        pltpu.sync_copy(acc_v, out_hbm.at[b])


def kernel(pred_class, pred_bbox, gt_class, gt_bbox):
    gt_class = gt_class.astype(jnp.int32)
    gt_bboxT = jnp.transpose(gt_bbox, (0, 2, 1))
    gt_class_r = gt_class[:, None, :]
    cost, lossp, extra = _dense_call(
        pred_class, pred_bbox, gt_bboxT, gt_class_r, gt_class)
    parts = _get_match_kernel()(cost.reshape(B, QG), lossp.reshape(B, QG))
    return jnp.sum(parts) + jnp.sum(extra)


# speedup vs baseline: 5.8891x; 1.2128x over previous
import functools

import jax
import jax.numpy as jnp
from jax import lax
from jax.experimental import pallas as pl
from jax.experimental.pallas import tpu as pltpu
from jax.experimental.pallas import tpu_sc as plsc

NUM_CLASSES = 91
C1 = NUM_CLASSES + 1
EOS_COEF = 0.1
W_CLASS, W_BBOX, W_GIOU = 1.0, 5.0, 2.0
EPS = 1e-7
B, Q, G = 64, 300, 50
LANES = 16
GPAD = 56
QROW = 384
NCHUNK = QROW // LANES
ROWELEMS = GPAD * QROW


def _dense_body(pc_ref, pbt_ref, gb_ref, gcc_ref, gcall_ref,
                cost_ref, lossp_ref):
    pc = pc_ref[0]
    pbt = pbt_ref[0]
    gb = gb_ref[0]
    gcc = gcc_ref[0]
    gcall = gcall_ref[...]

    pct = jnp.swapaxes(pc, 0, 1)
    m = jnp.max(pct, axis=0, keepdims=True)
    e = jnp.exp(pct - m)
    lse = jnp.log(jnp.sum(e, axis=0, keepdims=True)) + m
    logp0 = pct[0:1, :] - lse

    cls_iota = lax.broadcasted_iota(jnp.int32, (G, C1), 1)
    oht = (cls_iota == gcc).astype(jnp.float32)
    pcg = jnp.dot(oht, pct, preferred_element_type=jnp.float32)
    logp_g = pcg - lse
    prob_g = jnp.exp(logp_g)

    px0, py0, px1, py1 = pbt[0:1, :], pbt[1:2, :], pbt[2:3, :], pbt[3:4, :]
    gx0, gy0, gx1, gy1 = gb[:, 0:1], gb[:, 1:2], gb[:, 2:3], gb[:, 3:4]
    l1 = (jnp.abs(px0 - gx0) + jnp.abs(py0 - gy0)
          + jnp.abs(px1 - gx1) + jnp.abs(py1 - gy1))
    ix1 = jnp.maximum(px0, gx0)
    iy1 = jnp.maximum(py0, gy0)
    ix2 = jnp.minimum(px1, gx1)
    iy2 = jnp.minimum(py1, gy1)
    inter = jnp.maximum(ix2 - ix1, 0.0) * jnp.maximum(iy2 - iy1, 0.0)
    a1 = (px1 - px0) * (py1 - py0)
    a2 = (gx1 - gx0) * (gy1 - gy0)
    union = a1 + a2 - inter
    iou = inter / (union + EPS)
    ex1 = jnp.minimum(px0, gx0)
    ey1 = jnp.minimum(py0, gy0)
    ex2 = jnp.maximum(px1, gx1)
    ey2 = jnp.maximum(py1, gy1)
    enc = (ex2 - ex1) * (ey2 - ey1)
    giou = iou - (enc - union) / (enc + EPS)

    cost = -W_CLASS * prob_g + W_BBOX * l1 - W_GIOU * giou

    nvalid = jnp.sum((gcall != 0).astype(jnp.float32))
    class_den = EOS_COEF * (B * Q) + (1.0 - EOS_COEF) * nvalid
    num_objs = jnp.maximum(nvalid, 1.0)

    wg = jnp.where(gcc == 0, EOS_COEF, 1.0).astype(jnp.float32)
    validg = (gcc != 0).astype(jnp.float32)
    lossp = (W_CLASS * (wg * (-logp_g) + EOS_COEF * logp0) / class_den
             + validg * (W_BBOX * l1 + W_GIOU * (1.0 - giou)) / num_objs)
    extra = W_CLASS * EOS_COEF * (-jnp.sum(logp0)) / class_den
    row0 = lax.broadcasted_iota(jnp.int32, (G, Q), 0) == 0
    lossp = lossp + jnp.where(row0, extra, 0.0)

    cost_ref[0, 0:G, 0:Q] = cost
    cost_ref[0, 0:G, Q:QROW] = jnp.full((G, QROW - Q), jnp.inf, jnp.float32)
    lossp_ref[0, 0:G, 0:Q] = lossp


def _dense_call(pred_class, pred_bboxT, gt_bbox, gt_class_col, gt_class):
    return pl.pallas_call(
        _dense_body,
        grid=(B,),
        in_specs=[
            pl.BlockSpec((1, Q, C1), lambda b: (b, 0, 0)),
            pl.BlockSpec((1, 4, Q), lambda b: (b, 0, 0)),
            pl.BlockSpec((1, G, 4), lambda b: (b, 0, 0)),
            pl.BlockSpec((1, G, 1), lambda b: (b, 0, 0)),
            pl.BlockSpec((B, G), lambda b: (0, 0)),
        ],
        out_specs=[
            pl.BlockSpec((1, GPAD, QROW), lambda b: (b, 0, 0)),
            pl.BlockSpec((1, GPAD, QROW), lambda b: (b, 0, 0)),
        ],
        out_shape=[
            jax.ShapeDtypeStruct((B, GPAD, QROW), jnp.float32),
            jax.ShapeDtypeStruct((B, GPAD, QROW), jnp.float32),
        ],
    )(pred_class, pred_bboxT, gt_bbox, gt_class_col, gt_class)


@functools.lru_cache(maxsize=1)
def _get_match_kernel():
    mesh = plsc.VectorSubcoreMesh(core_axis_name="c", subcore_axis_name="s",
                                  num_cores=2, num_subcores=16)
    return functools.partial(
        pl.kernel,
        out_type=jax.ShapeDtypeStruct((B, LANES), jnp.float32),
        mesh=mesh,
        scratch_types=[
            pltpu.VMEM((ROWELEMS,), jnp.float32),
            pltpu.VMEM((ROWELEMS,), jnp.float32),
            pltpu.VMEM((QROW,), jnp.float32),
            pltpu.VMEM((LANES,), jnp.float32),
        ],
        compiler_params=pltpu.CompilerParams(needs_layout_passes=False),
    )(_match_body)


def _match_body(cost_hbm, lossp_hbm, out_hbm, cost_v, lossp_v, pen_v, acc_v):
    wid = lax.axis_index("s") * 2 + lax.axis_index("c")
    lane = lax.iota(jnp.int32, LANES)
    lane0 = lane == 0
    zeros = jnp.zeros((LANES,), jnp.float32)
    ones = jnp.ones((LANES,), jnp.float32)

    for i in range(2):
        b = wid * 2 + i
        pltpu.sync_copy(cost_hbm.at[b], cost_v)
        pltpu.sync_copy(lossp_hbm.at[b], lossp_v)
        for k in range(NCHUNK):
            pen_v[pl.ds(k * LANES, LANES)] = zeros

        def gbody(g, acc):
            base = g * QROW
            mvec = jnp.full((LANES,), jnp.inf, jnp.float32)
            kidx = jnp.zeros((LANES,), jnp.int32)
            for k in range(NCHUNK):
                vals = cost_v[pl.ds(base + k * LANES, LANES)]
                pen = pen_v[pl.ds(k * LANES, LANES)]
                aug = jnp.where(pen > 0.5, jnp.inf, vals)
                better = aug < mvec
                mvec = jnp.where(better, aug, mvec)
                kidx = jnp.where(better, k, kidx)
            mval = jnp.min(mvec)
            cand = jnp.where(mvec == mval, kidx * LANES + lane,
                             jnp.int32(1 << 30))
            q = jnp.min(cand)
            plsc.store_scatter(pen_v, [jnp.full((LANES,), q, jnp.int32)],
                               ones, mask=lane0)
            li = jnp.full((LANES,), base + q, jnp.int32)
            lv = plsc.load_gather(lossp_v, [li])
            return acc + jnp.where(lane0, lv, 0.0)

        acc = lax.fori_loop(0, G, gbody, jnp.zeros((LANES,), jnp.float32))
        acc_v[...] = acc
        pltpu.sync_copy(acc_v, out_hbm.at[b])


def kernel(pred_class, pred_bbox, gt_class, gt_bbox):
    gt_class = gt_class.astype(jnp.int32)
    pred_bboxT = jnp.transpose(pred_bbox, (0, 2, 1))
    gt_class_col = gt_class[:, :, None]
    cost, lossp = _dense_call(
        pred_class, pred_bboxT, gt_bbox, gt_class_col, gt_class)
    parts = _get_match_kernel()(cost.reshape(B, ROWELEMS),
                                lossp.reshape(B, ROWELEMS))
    return jnp.sum(parts)


# speedup vs baseline: 5.9619x; 1.0124x over previous
import functools

import jax
import jax.numpy as jnp
from jax import lax
from jax.experimental import pallas as pl
from jax.experimental.pallas import tpu as pltpu
from jax.experimental.pallas import tpu_sc as plsc

NUM_CLASSES = 91
C1 = NUM_CLASSES + 1
EOS_COEF = 0.1
W_CLASS, W_BBOX, W_GIOU = 1.0, 5.0, 2.0
EPS = 1e-7
B, Q, G = 64, 300, 50
LANES = 16
GPAD = 56
QROW = 384
NCHUNK = QROW // LANES
ROWELEMS = GPAD * QROW


def _dense_body(pc_ref, pb_ref, gb_ref, gcc_ref, gcall_ref,
                cost_ref, lossp_ref):
    pc = pc_ref[0]
    pbt = jnp.swapaxes(pb_ref[0], 0, 1)
    gb = gb_ref[0]
    gcc = gcc_ref[0]
    gcall = gcall_ref[...]

    pct = jnp.swapaxes(pc, 0, 1)
    m = jnp.max(pct, axis=0, keepdims=True)
    e = jnp.exp(pct - m)
    lse = jnp.log(jnp.sum(e, axis=0, keepdims=True)) + m
    logp0 = pct[0:1, :] - lse

    cls_iota = lax.broadcasted_iota(jnp.int32, (G, C1), 1)
    oht = (cls_iota == gcc).astype(jnp.float32)
    pcg = jnp.dot(oht, pct, preferred_element_type=jnp.float32)
    logp_g = pcg - lse
    prob_g = jnp.exp(logp_g)

    px0, py0, px1, py1 = pbt[0:1, :], pbt[1:2, :], pbt[2:3, :], pbt[3:4, :]
    gx0, gy0, gx1, gy1 = gb[:, 0:1], gb[:, 1:2], gb[:, 2:3], gb[:, 3:4]
    l1 = (jnp.abs(px0 - gx0) + jnp.abs(py0 - gy0)
          + jnp.abs(px1 - gx1) + jnp.abs(py1 - gy1))
    ix1 = jnp.maximum(px0, gx0)
    iy1 = jnp.maximum(py0, gy0)
    ix2 = jnp.minimum(px1, gx1)
    iy2 = jnp.minimum(py1, gy1)
    inter = jnp.maximum(ix2 - ix1, 0.0) * jnp.maximum(iy2 - iy1, 0.0)
    a1 = (px1 - px0) * (py1 - py0)
    a2 = (gx1 - gx0) * (gy1 - gy0)
    union = a1 + a2 - inter
    iou = inter / (union + EPS)
    ex1 = jnp.minimum(px0, gx0)
    ey1 = jnp.minimum(py0, gy0)
    ex2 = jnp.maximum(px1, gx1)
    ey2 = jnp.maximum(py1, gy1)
    enc = (ex2 - ex1) * (ey2 - ey1)
    giou = iou - (enc - union) / (enc + EPS)

    cost = -W_CLASS * prob_g + W_BBOX * l1 - W_GIOU * giou

    nvalid = jnp.sum((gcall != 0).astype(jnp.float32))
    class_den = EOS_COEF * (B * Q) + (1.0 - EOS_COEF) * nvalid
    num_objs = jnp.maximum(nvalid, 1.0)

    wg = jnp.where(gcc == 0, EOS_COEF, 1.0).astype(jnp.float32)
    validg = (gcc != 0).astype(jnp.float32)
    lossp = (W_CLASS * (wg * (-logp_g) + EOS_COEF * logp0) / class_den
             + validg * (W_BBOX * l1 + W_GIOU * (1.0 - giou)) / num_objs)
    extra = W_CLASS * EOS_COEF * (-jnp.sum(logp0)) / class_den
    row0 = lax.broadcasted_iota(jnp.int32, (G, Q), 0) == 0
    lossp = lossp + jnp.where(row0, extra, 0.0)

    cost_ref[0, 0:G, 0:Q] = cost
    cost_ref[0, 0:G, Q:QROW] = jnp.full((G, QROW - Q), jnp.inf, jnp.float32)
    lossp_ref[0, 0:G, 0:Q] = lossp


def _dense_call(pred_class, pred_bbox, gt_bbox, gt_class_col, gt_class):
    return pl.pallas_call(
        _dense_body,
        grid=(B,),
        in_specs=[
            pl.BlockSpec((1, Q, C1), lambda b: (b, 0, 0)),
            pl.BlockSpec((1, Q, 4), lambda b: (b, 0, 0)),
            pl.BlockSpec((1, G, 4), lambda b: (b, 0, 0)),
            pl.BlockSpec((1, G, 1), lambda b: (b, 0, 0)),
            pl.BlockSpec((B, G), lambda b: (0, 0)),
        ],
        out_specs=[
            pl.BlockSpec((1, GPAD, QROW), lambda b: (b, 0, 0)),
            pl.BlockSpec((1, GPAD, QROW), lambda b: (b, 0, 0)),
        ],
        out_shape=[
            jax.ShapeDtypeStruct((B, GPAD, QROW), jnp.float32),
            jax.ShapeDtypeStruct((B, GPAD, QROW), jnp.float32),
        ],
    )(pred_class, pred_bbox, gt_bbox, gt_class_col, gt_class)


@functools.lru_cache(maxsize=1)
def _get_match_kernel():
    mesh = plsc.VectorSubcoreMesh(core_axis_name="c", subcore_axis_name="s",
                                  num_cores=2, num_subcores=16)
    return functools.partial(
        pl.kernel,
        out_type=jax.ShapeDtypeStruct((B, LANES), jnp.float32),
        mesh=mesh,
        scratch_types=[
            pltpu.VMEM((GPAD, QROW), jnp.float32),
            pltpu.VMEM((GPAD, QROW), jnp.float32),
            pltpu.VMEM((QROW,), jnp.float32),
            pltpu.VMEM((LANES,), jnp.float32),
        ],
        compiler_params=pltpu.CompilerParams(needs_layout_passes=False),
    )(_match_body)


def _match_body(cost_hbm, lossp_hbm, out_hbm, cost_v, lossp_v, pen_v, acc_v):
    wid = lax.axis_index("s") * 2 + lax.axis_index("c")
    lane = lax.iota(jnp.int32, LANES)
    lane0 = lane == 0
    zeros = jnp.zeros((LANES,), jnp.float32)
    ones = jnp.ones((LANES,), jnp.float32)

    for i in range(2):
        b = wid * 2 + i
        pltpu.sync_copy(cost_hbm.at[b], cost_v)
        pltpu.sync_copy(lossp_hbm.at[b], lossp_v)
        for k in range(NCHUNK):
            pen_v[pl.ds(k * LANES, LANES)] = zeros

        def gbody(g, acc):
            mvec = jnp.full((LANES,), jnp.inf, jnp.float32)
            kidx = jnp.zeros((LANES,), jnp.int32)
            for k in range(NCHUNK):
                vals = cost_v[g, pl.ds(k * LANES, LANES)]
                pen = pen_v[pl.ds(k * LANES, LANES)]
                aug = jnp.where(pen > 0.5, jnp.inf, vals)
                better = aug < mvec
                mvec = jnp.where(better, aug, mvec)
                kidx = jnp.where(better, k, kidx)
            mval = jnp.min(mvec)
            cand = jnp.where(mvec == mval, kidx * LANES + lane,
                             jnp.int32(1 << 30))
            q = jnp.min(cand)
            plsc.store_scatter(pen_v, [jnp.full((LANES,), q, jnp.int32)],
                               ones, mask=lane0)
            lv = plsc.load_gather(lossp_v, [jnp.full((LANES,), g, jnp.int32),
                                            jnp.full((LANES,), q, jnp.int32)])
            return acc + jnp.where(lane0, lv, 0.0)

        acc = lax.fori_loop(0, G, gbody, jnp.zeros((LANES,), jnp.float32))
        acc_v[...] = acc
        pltpu.sync_copy(acc_v, out_hbm.at[b])


def kernel(pred_class, pred_bbox, gt_class, gt_bbox):
    gt_class = gt_class.astype(jnp.int32)
    gt_class_col = gt_class[:, :, None]
    cost, lossp = _dense_call(
        pred_class, pred_bbox, gt_bbox, gt_class_col, gt_class)
    parts = _get_match_kernel()(cost, lossp)
    return jnp.sum(parts)


# speedup vs baseline: 7.7539x; 1.3006x over previous
import functools

import jax
import jax.numpy as jnp
from jax import lax
from jax.experimental import pallas as pl
from jax.experimental.pallas import tpu as pltpu
from jax.experimental.pallas import tpu_sc as plsc

NUM_CLASSES = 91
C1 = NUM_CLASSES + 1
EOS_COEF = 0.1
W_CLASS, W_BBOX, W_GIOU = 1.0, 5.0, 2.0
EPS = 1e-7
B, Q, G = 64, 300, 50
LANES = 16
GPAD = 56
QROW = 384
NCHUNK = QROW // LANES
ROWELEMS = GPAD * QROW


BSTEP = 8


def _dense_body(pc_ref, pb_ref, gb_ref, gcc_ref, gcall_ref,
                cost_ref, lossp_ref):
    gcall = gcall_ref[...]
    nvalid = jnp.sum((gcall != 0).astype(jnp.float32))
    class_den = EOS_COEF * (B * Q) + (1.0 - EOS_COEF) * nvalid
    num_objs = jnp.maximum(nvalid, 1.0)
    infpad = jnp.full((G, QROW - Q), jnp.inf, jnp.float32)
    row0 = lax.broadcasted_iota(jnp.int32, (G, Q), 0) == 0
    cls_iota = lax.broadcasted_iota(jnp.int32, (G, C1), 1)

    for j in range(BSTEP):
        pc = pc_ref[j]
        pbt = jnp.swapaxes(pb_ref[j], 0, 1)
        gb = gb_ref[j]
        gcc = gcc_ref[j]

        pct = jnp.swapaxes(pc, 0, 1)
        m = jnp.max(pct, axis=0, keepdims=True)
        e = jnp.exp(pct - m)
        lse = jnp.log(jnp.sum(e, axis=0, keepdims=True)) + m
        logp0 = pct[0:1, :] - lse

        oht = (cls_iota == gcc).astype(jnp.float32)
        pcg = jnp.dot(oht, pct, preferred_element_type=jnp.float32)
        logp_g = pcg - lse
        prob_g = jnp.exp(logp_g)

        px0, py0, px1, py1 = pbt[0:1, :], pbt[1:2, :], pbt[2:3, :], pbt[3:4, :]
        gx0, gy0, gx1, gy1 = gb[:, 0:1], gb[:, 1:2], gb[:, 2:3], gb[:, 3:4]
        l1 = (jnp.abs(px0 - gx0) + jnp.abs(py0 - gy0)
              + jnp.abs(px1 - gx1) + jnp.abs(py1 - gy1))
        ix1 = jnp.maximum(px0, gx0)
        iy1 = jnp.maximum(py0, gy0)
        ix2 = jnp.minimum(px1, gx1)
        iy2 = jnp.minimum(py1, gy1)
        inter = jnp.maximum(ix2 - ix1, 0.0) * jnp.maximum(iy2 - iy1, 0.0)
        a1 = (px1 - px0) * (py1 - py0)
        a2 = (gx1 - gx0) * (gy1 - gy0)
        union = a1 + a2 - inter
        iou = inter / (union + EPS)
        ex1 = jnp.minimum(px0, gx0)
        ey1 = jnp.minimum(py0, gy0)
        ex2 = jnp.maximum(px1, gx1)
        ey2 = jnp.maximum(py1, gy1)
        enc = (ex2 - ex1) * (ey2 - ey1)
        giou = iou - (enc - union) / (enc + EPS)

        cost = -W_CLASS * prob_g + W_BBOX * l1 - W_GIOU * giou

        wg = jnp.where(gcc == 0, EOS_COEF, 1.0).astype(jnp.float32)
        validg = (gcc != 0).astype(jnp.float32)
        lossp = (W_CLASS * (wg * (-logp_g) + EOS_COEF * logp0) / class_den
                 + validg * (W_BBOX * l1 + W_GIOU * (1.0 - giou)) / num_objs)
        extra = W_CLASS * EOS_COEF * (-jnp.sum(logp0)) / class_den
        lossp = lossp + jnp.where(row0, extra, 0.0)

        cost_ref[j, 0:G, 0:Q] = cost
        cost_ref[j, 0:G, Q:QROW] = infpad
        lossp_ref[j, 0:G, 0:Q] = lossp


def _dense_call(pred_class, pred_bbox, gt_bbox, gt_class_col, gt_class):
    return pl.pallas_call(
        _dense_body,
        grid=(B // BSTEP,),
        in_specs=[
            pl.BlockSpec((BSTEP, Q, C1), lambda b: (b, 0, 0)),
            pl.BlockSpec((BSTEP, Q, 4), lambda b: (b, 0, 0)),
            pl.BlockSpec((BSTEP, G, 4), lambda b: (b, 0, 0)),
            pl.BlockSpec((BSTEP, G, 1), lambda b: (b, 0, 0)),
            pl.BlockSpec((B, G), lambda b: (0, 0)),
        ],
        out_specs=[
            pl.BlockSpec((BSTEP, GPAD, QROW), lambda b: (b, 0, 0)),
            pl.BlockSpec((BSTEP, GPAD, QROW), lambda b: (b, 0, 0)),
        ],
        out_shape=[
            jax.ShapeDtypeStruct((B, GPAD, QROW), jnp.float32),
            jax.ShapeDtypeStruct((B, GPAD, QROW), jnp.float32),
        ],
        compiler_params=pltpu.CompilerParams(
            dimension_semantics=("parallel",)),
    )(pred_class, pred_bbox, gt_bbox, gt_class_col, gt_class)


@functools.lru_cache(maxsize=1)
def _get_match_kernel():
    mesh = plsc.VectorSubcoreMesh(core_axis_name="c", subcore_axis_name="s",
                                  num_cores=2, num_subcores=16)
    return functools.partial(
        pl.kernel,
        out_type=jax.ShapeDtypeStruct((B, LANES), jnp.float32),
        mesh=mesh,
        scratch_types=[
            pltpu.VMEM((GPAD, QROW), jnp.float32),
            pltpu.VMEM((GPAD, QROW), jnp.float32),
            pltpu.VMEM((QROW,), jnp.float32),
            pltpu.VMEM((LANES,), jnp.float32),
        ],
        compiler_params=pltpu.CompilerParams(needs_layout_passes=False),
    )(_match_body)


def _match_body(cost_hbm, lossp_hbm, out_hbm, cost_v, lossp_v, pen_v, acc_v):
    wid = lax.axis_index("s") * 2 + lax.axis_index("c")
    lane = lax.iota(jnp.int32, LANES)
    lane0 = lane == 0
    zeros = jnp.zeros((LANES,), jnp.float32)
    ones = jnp.ones((LANES,), jnp.float32)

    for i in range(2):
        b = wid * 2 + i
        pltpu.sync_copy(cost_hbm.at[b], cost_v)
        pltpu.sync_copy(lossp_hbm.at[b], lossp_v)
        for k in range(NCHUNK):
            pen_v[pl.ds(k * LANES, LANES)] = zeros

        def gbody(g, acc):
            mvec = jnp.full((LANES,), jnp.inf, jnp.float32)
            kidx = jnp.zeros((LANES,), jnp.int32)
            for k in range(NCHUNK):
                vals = cost_v[g, pl.ds(k * LANES, LANES)]
                pen = pen_v[pl.ds(k * LANES, LANES)]
                aug = jnp.where(pen > 0.5, jnp.inf, vals)
                better = aug < mvec
                mvec = jnp.where(better, aug, mvec)
                kidx = jnp.where(better, k, kidx)
            mval = jnp.min(mvec)
            cand = jnp.where(mvec == mval, kidx * LANES + lane,
                             jnp.int32(1 << 30))
            q = jnp.min(cand)
            plsc.store_scatter(pen_v, [jnp.full((LANES,), q, jnp.int32)],
                               ones, mask=lane0)
            lv = plsc.load_gather(lossp_v, [jnp.full((LANES,), g, jnp.int32),
                                            jnp.full((LANES,), q, jnp.int32)])
            return acc + jnp.where(lane0, lv, 0.0)

        acc = lax.fori_loop(0, G, gbody, jnp.zeros((LANES,), jnp.float32))
        acc_v[...] = acc
        pltpu.sync_copy(acc_v, out_hbm.at[b])


def kernel(pred_class, pred_bbox, gt_class, gt_bbox):
    gt_class = gt_class.astype(jnp.int32)
    gt_class_col = gt_class[:, :, None]
    cost, lossp = _dense_call(
        pred_class, pred_bbox, gt_bbox, gt_class_col, gt_class)
    parts = _get_match_kernel()(cost, lossp)
    return jnp.sum(parts)


# speedup vs baseline: 7.9961x; 1.0312x over previous
import functools

import jax
import jax.numpy as jnp
from jax import lax
from jax.experimental import pallas as pl
from jax.experimental.pallas import tpu as pltpu
from jax.experimental.pallas import tpu_sc as plsc

NUM_CLASSES = 91
C1 = NUM_CLASSES + 1
EOS_COEF = 0.1
W_CLASS, W_BBOX, W_GIOU = 1.0, 5.0, 2.0
EPS = 1e-7
B, Q, G = 64, 300, 50
LANES = 16
GPAD = 56
QROW = 384
NCHUNK = QROW // LANES
ROWELEMS = GPAD * QROW


BSTEP = 8
HB = B // 2


def _dense_body(pc_ref, pb_ref, gb_ref, gcc_ref, gcall_ref,
                cost_ref, lossp_ref):
    gcall = gcall_ref[...]
    nvalid = jnp.sum((gcall != 0).astype(jnp.float32))
    class_den = EOS_COEF * (B * Q) + (1.0 - EOS_COEF) * nvalid
    num_objs = jnp.maximum(nvalid, 1.0)
    infpad = jnp.full((G, QROW - Q), jnp.inf, jnp.float32)
    row0 = lax.broadcasted_iota(jnp.int32, (G, Q), 0) == 0
    cls_iota = lax.broadcasted_iota(jnp.int32, (G, C1), 1)

    for j in range(BSTEP):
        pc = pc_ref[j]
        pbt = jnp.swapaxes(pb_ref[j], 0, 1)
        gb = gb_ref[j]
        gcc = gcc_ref[j]

        pct = jnp.swapaxes(pc, 0, 1)
        m = jnp.max(pct, axis=0, keepdims=True)
        e = jnp.exp(pct - m)
        lse = jnp.log(jnp.sum(e, axis=0, keepdims=True)) + m
        logp0 = pct[0:1, :] - lse

        oht = (cls_iota == gcc).astype(jnp.float32)
        pcg = jnp.dot(oht, pct, preferred_element_type=jnp.float32)
        logp_g = pcg - lse
        prob_g = jnp.exp(logp_g)

        px0, py0, px1, py1 = pbt[0:1, :], pbt[1:2, :], pbt[2:3, :], pbt[3:4, :]
        gx0, gy0, gx1, gy1 = gb[:, 0:1], gb[:, 1:2], gb[:, 2:3], gb[:, 3:4]
        l1 = (jnp.abs(px0 - gx0) + jnp.abs(py0 - gy0)
              + jnp.abs(px1 - gx1) + jnp.abs(py1 - gy1))
        ix1 = jnp.maximum(px0, gx0)
        iy1 = jnp.maximum(py0, gy0)
        ix2 = jnp.minimum(px1, gx1)
        iy2 = jnp.minimum(py1, gy1)
        inter = jnp.maximum(ix2 - ix1, 0.0) * jnp.maximum(iy2 - iy1, 0.0)
        a1 = (px1 - px0) * (py1 - py0)
        a2 = (gx1 - gx0) * (gy1 - gy0)
        union = a1 + a2 - inter
        iou = inter / (union + EPS)
        ex1 = jnp.minimum(px0, gx0)
        ey1 = jnp.minimum(py0, gy0)
        ex2 = jnp.maximum(px1, gx1)
        ey2 = jnp.maximum(py1, gy1)
        enc = (ex2 - ex1) * (ey2 - ey1)
        giou = iou - (enc - union) / (enc + EPS)

        cost = -W_CLASS * prob_g + W_BBOX * l1 - W_GIOU * giou

        wg = jnp.where(gcc == 0, EOS_COEF, 1.0).astype(jnp.float32)
        validg = (gcc != 0).astype(jnp.float32)
        lossp = (W_CLASS * (wg * (-logp_g) + EOS_COEF * logp0) / class_den
                 + validg * (W_BBOX * l1 + W_GIOU * (1.0 - giou)) / num_objs)
        extra = W_CLASS * EOS_COEF * (-jnp.sum(logp0)) / class_den
        lossp = lossp + jnp.where(row0, extra, 0.0)

        cost_ref[j, 0:G, 0:Q] = cost
        cost_ref[j, 0:G, Q:QROW] = infpad
        lossp_ref[j, 0:G, 0:Q] = lossp


def _dense_call(pred_class, pred_bbox, gt_bbox, gt_class_col, gt_class, phase):
    off = phase * (HB // BSTEP)
    return pl.pallas_call(
        _dense_body,
        grid=(HB // BSTEP,),
        in_specs=[
            pl.BlockSpec((BSTEP, Q, C1), lambda b: (b + off, 0, 0)),
            pl.BlockSpec((BSTEP, Q, 4), lambda b: (b + off, 0, 0)),
            pl.BlockSpec((BSTEP, G, 4), lambda b: (b + off, 0, 0)),
            pl.BlockSpec((BSTEP, G, 1), lambda b: (b + off, 0, 0)),
            pl.BlockSpec((B, G), lambda b: (0, 0)),
        ],
        out_specs=[
            pl.BlockSpec((BSTEP, GPAD, QROW), lambda b: (b, 0, 0)),
            pl.BlockSpec((BSTEP, GPAD, QROW), lambda b: (b, 0, 0)),
        ],
        out_shape=[
            jax.ShapeDtypeStruct((HB, GPAD, QROW), jnp.float32),
            jax.ShapeDtypeStruct((HB, GPAD, QROW), jnp.float32),
        ],
        compiler_params=pltpu.CompilerParams(
            dimension_semantics=("parallel",)),
    )(pred_class, pred_bbox, gt_bbox, gt_class_col, gt_class)


@functools.lru_cache(maxsize=1)
def _get_match_kernel():
    mesh = plsc.VectorSubcoreMesh(core_axis_name="c", subcore_axis_name="s",
                                  num_cores=2, num_subcores=16)
    return functools.partial(
        pl.kernel,
        out_type=jax.ShapeDtypeStruct((HB, LANES), jnp.float32),
        mesh=mesh,
        scratch_types=[
            pltpu.VMEM((GPAD, QROW), jnp.float32),
            pltpu.VMEM((GPAD, QROW), jnp.float32),
            pltpu.VMEM((QROW,), jnp.float32),
            pltpu.VMEM((LANES,), jnp.float32),
        ],
        compiler_params=pltpu.CompilerParams(needs_layout_passes=False),
    )(_match_body)


def _match_body(cost_hbm, lossp_hbm, out_hbm, cost_v, lossp_v, pen_v, acc_v):
    wid = lax.axis_index("s") * 2 + lax.axis_index("c")
    lane = lax.iota(jnp.int32, LANES)
    lane0 = lane == 0
    zeros = jnp.zeros((LANES,), jnp.float32)
    ones = jnp.ones((LANES,), jnp.float32)

    for i in range(HB // 32):
        b = wid + 32 * i
        pltpu.sync_copy(cost_hbm.at[b], cost_v)
        pltpu.sync_copy(lossp_hbm.at[b], lossp_v)
        for k in range(NCHUNK):
            pen_v[pl.ds(k * LANES, LANES)] = zeros

        def gbody(g, acc):
            mvec = jnp.full((LANES,), jnp.inf, jnp.float32)
            kidx = jnp.zeros((LANES,), jnp.int32)
            for k in range(NCHUNK):
                vals = cost_v[g, pl.ds(k * LANES, LANES)]
                pen = pen_v[pl.ds(k * LANES, LANES)]
                aug = jnp.where(pen > 0.5, jnp.inf, vals)
                better = aug < mvec
                mvec = jnp.where(better, aug, mvec)
                kidx = jnp.where(better, k, kidx)
            mval = jnp.min(mvec)
            cand = jnp.where(mvec == mval, kidx * LANES + lane,
                             jnp.int32(1 << 30))
            q = jnp.min(cand)
            plsc.store_scatter(pen_v, [jnp.full((LANES,), q, jnp.int32)],
                               ones, mask=lane0)
            lv = plsc.load_gather(lossp_v, [jnp.full((LANES,), g, jnp.int32),
                                            jnp.full((LANES,), q, jnp.int32)])
            return acc + jnp.where(lane0, lv, 0.0)

        acc = lax.fori_loop(0, G, gbody, jnp.zeros((LANES,), jnp.float32))
        acc_v[...] = acc
        pltpu.sync_copy(acc_v, out_hbm.at[b])


def kernel(pred_class, pred_bbox, gt_class, gt_bbox):
    gt_class = gt_class.astype(jnp.int32)
    gt_class_col = gt_class[:, :, None]
    match = _get_match_kernel()
    cost0, lossp0 = _dense_call(
        pred_class, pred_bbox, gt_bbox, gt_class_col, gt_class, 0)
    parts0 = match(cost0, lossp0)
    cost1, lossp1 = _dense_call(
        pred_class, pred_bbox, gt_bbox, gt_class_col, gt_class, 1)
    parts1 = match(cost1, lossp1)
    return jnp.sum(parts0) + jnp.sum(parts1)


# speedup vs baseline: 8.0287x; 1.0041x over previous
import functools

import jax
import jax.numpy as jnp
from jax import lax
from jax.experimental import pallas as pl
from jax.experimental.pallas import tpu as pltpu
from jax.experimental.pallas import tpu_sc as plsc

NUM_CLASSES = 91
C1 = NUM_CLASSES + 1
EOS_COEF = 0.1
W_CLASS, W_BBOX, W_GIOU = 1.0, 5.0, 2.0
EPS = 1e-7
B, Q, G = 64, 300, 50
LANES = 16
GPAD = 56
QROW = 384
NCHUNK = QROW // LANES
NSCAN = 19
ROWELEMS = GPAD * QROW


BSTEP = 8
HB = B // 2


def _dense_body(pc_ref, pb_ref, gb_ref, gcc_ref, gcall_ref,
                cost_ref, lossp_ref):
    gcall = gcall_ref[...]
    nvalid = jnp.sum((gcall != 0).astype(jnp.float32))
    class_den = EOS_COEF * (B * Q) + (1.0 - EOS_COEF) * nvalid
    num_objs = jnp.maximum(nvalid, 1.0)
    infpad = jnp.full((G, QROW - Q), jnp.inf, jnp.float32)
    row0 = lax.broadcasted_iota(jnp.int32, (G, Q), 0) == 0
    cls_iota = lax.broadcasted_iota(jnp.int32, (G, C1), 1)

    for j in range(BSTEP):
        pc = pc_ref[j]
        pbt = jnp.swapaxes(pb_ref[j], 0, 1)
        gb = gb_ref[j]
        gcc = gcc_ref[j]

        pct = jnp.swapaxes(pc, 0, 1)
        m = jnp.max(pct, axis=0, keepdims=True)
        e = jnp.exp(pct - m)
        lse = jnp.log(jnp.sum(e, axis=0, keepdims=True)) + m
        logp0 = pct[0:1, :] - lse

        oht = (cls_iota == gcc).astype(jnp.float32)
        pcg = jnp.dot(oht, pct, preferred_element_type=jnp.float32)
        logp_g = pcg - lse
        prob_g = jnp.exp(logp_g)

        px0, py0, px1, py1 = pbt[0:1, :], pbt[1:2, :], pbt[2:3, :], pbt[3:4, :]
        gx0, gy0, gx1, gy1 = gb[:, 0:1], gb[:, 1:2], gb[:, 2:3], gb[:, 3:4]
        l1 = (jnp.abs(px0 - gx0) + jnp.abs(py0 - gy0)
              + jnp.abs(px1 - gx1) + jnp.abs(py1 - gy1))
        ix1 = jnp.maximum(px0, gx0)
        iy1 = jnp.maximum(py0, gy0)
        ix2 = jnp.minimum(px1, gx1)
        iy2 = jnp.minimum(py1, gy1)
        inter = jnp.maximum(ix2 - ix1, 0.0) * jnp.maximum(iy2 - iy1, 0.0)
        a1 = (px1 - px0) * (py1 - py0)
        a2 = (gx1 - gx0) * (gy1 - gy0)
        union = a1 + a2 - inter
        iou = inter / (union + EPS)
        ex1 = jnp.minimum(px0, gx0)
        ey1 = jnp.minimum(py0, gy0)
        ex2 = jnp.maximum(px1, gx1)
        ey2 = jnp.maximum(py1, gy1)
        enc = (ex2 - ex1) * (ey2 - ey1)
        giou = iou - (enc - union) / (enc + EPS)

        cost = -W_CLASS * prob_g + W_BBOX * l1 - W_GIOU * giou

        wg = jnp.where(gcc == 0, EOS_COEF, 1.0).astype(jnp.float32)
        validg = (gcc != 0).astype(jnp.float32)
        lossp = (W_CLASS * (wg * (-logp_g) + EOS_COEF * logp0) / class_den
                 + validg * (W_BBOX * l1 + W_GIOU * (1.0 - giou)) / num_objs)
        extra = W_CLASS * EOS_COEF * (-jnp.sum(logp0)) / class_den
        lossp = lossp + jnp.where(row0, extra, 0.0)

        cost_ref[j, 0:G, 0:Q] = cost
        cost_ref[j, 0:G, Q:QROW] = infpad
        lossp_ref[j, 0:G, 0:Q] = lossp


def _dense_call(pred_class, pred_bbox, gt_bbox, gt_class_col, gt_class, phase):
    off = phase * (HB // BSTEP)
    return pl.pallas_call(
        _dense_body,
        grid=(HB // BSTEP,),
        in_specs=[
            pl.BlockSpec((BSTEP, Q, C1), lambda b: (b + off, 0, 0)),
            pl.BlockSpec((BSTEP, Q, 4), lambda b: (b + off, 0, 0)),
            pl.BlockSpec((BSTEP, G, 4), lambda b: (b + off, 0, 0)),
            pl.BlockSpec((BSTEP, G, 1), lambda b: (b + off, 0, 0)),
            pl.BlockSpec((B, G), lambda b: (0, 0)),
        ],
        out_specs=[
            pl.BlockSpec((BSTEP, GPAD, QROW), lambda b: (b, 0, 0)),
            pl.BlockSpec((BSTEP, GPAD, QROW), lambda b: (b, 0, 0)),
        ],
        out_shape=[
            jax.ShapeDtypeStruct((HB, GPAD, QROW), jnp.float32),
            jax.ShapeDtypeStruct((HB, GPAD, QROW), jnp.float32),
        ],
        compiler_params=pltpu.CompilerParams(
            dimension_semantics=("parallel",)),
    )(pred_class, pred_bbox, gt_bbox, gt_class_col, gt_class)


@functools.lru_cache(maxsize=1)
def _get_match_kernel():
    mesh = plsc.VectorSubcoreMesh(core_axis_name="c", subcore_axis_name="s",
                                  num_cores=2, num_subcores=16)
    return functools.partial(
        pl.kernel,
        out_type=jax.ShapeDtypeStruct((HB, LANES), jnp.float32),
        mesh=mesh,
        scratch_types=[
            pltpu.VMEM((GPAD, QROW), jnp.float32),
            pltpu.VMEM((GPAD, QROW), jnp.float32),
            pltpu.VMEM((QROW,), jnp.float32),
            pltpu.VMEM((LANES,), jnp.float32),
        ],
        compiler_params=pltpu.CompilerParams(needs_layout_passes=False),
    )(_match_body)


def _match_body(cost_hbm, lossp_hbm, out_hbm, cost_v, lossp_v, pen_v, acc_v):
    wid = lax.axis_index("s") * 2 + lax.axis_index("c")
    lane = lax.iota(jnp.int32, LANES)
    lane0 = lane == 0
    zeros = jnp.zeros((LANES,), jnp.float32)
    ones = jnp.ones((LANES,), jnp.float32)

    for i in range(HB // 32):
        b = wid + 32 * i
        pltpu.sync_copy(cost_hbm.at[b], cost_v)
        pltpu.sync_copy(lossp_hbm.at[b], lossp_v)
        for k in range(NSCAN):
            pen_v[pl.ds(k * LANES, LANES)] = zeros

        def gbody(g, acc):
            mvec = jnp.full((LANES,), jnp.inf, jnp.float32)
            kidx = jnp.zeros((LANES,), jnp.int32)
            for k in range(NSCAN):
                vals = cost_v[g, pl.ds(k * LANES, LANES)]
                pen = pen_v[pl.ds(k * LANES, LANES)]
                aug = jnp.where(pen > 0.5, jnp.inf, vals)
                better = aug < mvec
                mvec = jnp.where(better, aug, mvec)
                kidx = jnp.where(better, k, kidx)
            mval = jnp.min(mvec)
            cand = jnp.where(mvec == mval, kidx * LANES + lane,
                             jnp.int32(1 << 30))
            q = jnp.min(cand)
            plsc.store_scatter(pen_v, [jnp.full((LANES,), q, jnp.int32)],
                               ones, mask=lane0)
            lv = plsc.load_gather(lossp_v, [jnp.full((LANES,), g, jnp.int32),
                                            jnp.full((LANES,), q, jnp.int32)])
            return acc + jnp.where(lane0, lv, 0.0)

        acc = lax.fori_loop(0, G, gbody, jnp.zeros((LANES,), jnp.float32))
        acc_v[...] = acc
        pltpu.sync_copy(acc_v, out_hbm.at[b])


def kernel(pred_class, pred_bbox, gt_class, gt_bbox):
    gt_class = gt_class.astype(jnp.int32)
    gt_class_col = gt_class[:, :, None]
    match = _get_match_kernel()
    cost0, lossp0 = _dense_call(
        pred_class, pred_bbox, gt_bbox, gt_class_col, gt_class, 0)
    parts0 = match(cost0, lossp0)
    cost1, lossp1 = _dense_call(
        pred_class, pred_bbox, gt_bbox, gt_class_col, gt_class, 1)
    parts1 = match(cost1, lossp1)
    return jnp.sum(parts0) + jnp.sum(parts1)
